# baseline (speedup 1.0000x reference)
import functools

import jax
import jax.numpy as jnp
from jax import lax
from jax.experimental import pallas as pl
from jax.experimental.pallas import tpu as pltpu
from jax.experimental.pallas import tpu_sc as plsc

H = 2048
I = 768
E = 64
K = 8
T = 2048
P = T * K
M = 512
NB = P // M + E
PP = NB * M


def _router_body(x_ref, gw_ref, w_ref, s_ref):
    x = x_ref[...]
    gw = gw_ref[...]
    logits = lax.dot_general(x, gw, (((1,), (1,)), ((), ())),
                             preferred_element_type=jnp.float32)
    m = jnp.max(logits, axis=1, keepdims=True)
    p = jnp.exp(logits - m)
    p = p / jnp.sum(p, axis=1, keepdims=True)
    iota = lax.broadcasted_iota(jnp.int32, p.shape, 1)
    vals = []
    idxs = []
    for _ in range(K):
        mk = jnp.max(p, axis=1, keepdims=True)
        amk = jnp.min(jnp.where(p == mk, iota, E), axis=1, keepdims=True)
        vals.append(mk)
        idxs.append(amk)
        p = jnp.where(iota == amk, -1.0, p)
    w8 = jnp.concatenate(vals, axis=1)
    s8 = jnp.concatenate(idxs, axis=1)
    w_ref[...] = w8 / jnp.sum(w8, axis=1, keepdims=True)
    s_ref[...] = s8


def _route(x, gate_w, interpret=False):
    RB = 256
    return pl.pallas_call(
        _router_body,
        grid=(T // RB,),
        in_specs=[
            pl.BlockSpec((RB, H), lambda b: (b, 0)),
            pl.BlockSpec((E, H), lambda b: (0, 0)),
        ],
        out_specs=[
            pl.BlockSpec((RB, K), lambda b: (b, 0)),
            pl.BlockSpec((RB, K), lambda b: (b, 0)),
        ],
        out_shape=[
            jax.ShapeDtypeStruct((T, K), jnp.float32),
            jax.ShapeDtypeStruct((T, K), jnp.int32),
        ],
        interpret=interpret,
    )(x, gate_w)


def _plan_body(sel_ref, pos_ref, be_ref, bv_ref):
    iot = lax.broadcasted_iota(jnp.int32, (T, E), 1)
    carry = jnp.zeros((1, E), jnp.float32)
    Os = []
    incls = []
    for k in range(K):
        col = sel_ref[:, k:k + 1]
        O = (col == iot).astype(jnp.float32)
        incl = O
        sh = 1
        while sh < T:
            incl = incl + jnp.concatenate(
                [jnp.zeros((sh, E), jnp.float32), incl[:T - sh]], axis=0)
            sh *= 2
        incl = incl + carry
        Os.append(O)
        incls.append(incl)
        carry = incl[T - 1:T, :]
    counts = carry
    nblk = jnp.floor((counts + (M - 1)) / M)
    trili = (lax.broadcasted_iota(jnp.int32, (E, E), 0)
             <= lax.broadcasted_iota(jnp.int32, (E, E), 1)).astype(jnp.float32)
    cumblk = lax.dot_general(nblk, trili, (((1,), (0,)), ((), ())),
                             preferred_element_type=jnp.float32)
    pad_start = (cumblk - nblk) * M
    for k in range(K):
        slot = jnp.sum((incls[k] - 1.0 + pad_start) * Os[k],
                       axis=1, keepdims=True)
        pos_ref[:, k:k + 1] = slot.astype(jnp.int32)
    total = jnp.sum(nblk)
    biot = lax.broadcasted_iota(jnp.int32, (NB, E), 0).astype(jnp.float32)
    cb = jnp.broadcast_to(cumblk, (NB, E))
    be_raw = jnp.sum((cb <= biot).astype(jnp.int32), axis=1, keepdims=True)
    be_c = jnp.minimum(be_raw, E - 1)
    bvec = lax.broadcasted_iota(jnp.int32, (NB, 1), 0).astype(jnp.float32)
    bv = (bvec < total).astype(jnp.int32)
    last_e = jnp.sum(jnp.where(bvec == total - 1.0, be_c, 0))
    be_ref[...] = jnp.where(bv == 1, be_c, last_e)
    bv_ref[...] = bv


def _plan(sel, interpret=False):
    return pl.pallas_call(
        _plan_body,
        out_shape=[
            jax.ShapeDtypeStruct((T, K), jnp.int32),
            jax.ShapeDtypeStruct((NB, 1), jnp.int32),
            jax.ShapeDtypeStruct((NB, 1), jnp.int32),
        ],
        interpret=interpret,
    )(sel)


_NW = 32
_GC = 32
_GN = PP // _NW // _GC


def _sc_mesh():
    return plsc.VectorSubcoreMesh(core_axis_name="c", subcore_axis_name="s")


def _make_gather():
    HW = H // 2

    @functools.partial(
        pl.kernel,
        out_type=jax.ShapeDtypeStruct((PP, HW), jnp.int32),
        mesh=_sc_mesh(),
        scratch_types=[
            pltpu.VMEM((_GC,), jnp.int32),
            pltpu.VMEM((_GC,), jnp.int32),
            pltpu.VMEM((_GC, HW), jnp.int32),
            pltpu.VMEM((_GC, HW), jnp.int32),
            pltpu.SemaphoreType.DMA,
            pltpu.SemaphoreType.DMA,
        ],
    )
    def gather_k(x_hbm, tok_hbm, xs_hbm, idx0, idx1, buf0, buf1, sem0, sem1):
        wid = lax.axis_index("s") * 2 + lax.axis_index("c")
        base = wid * (PP // _NW)
        idxs = (idx0, idx1)
        bufs = (buf0, buf1)
        sems = (sem0, sem1)
        copies = [None, None]
        for i in range(_GN):
            b = i % 2
            pltpu.sync_copy(tok_hbm.at[pl.ds(base + i * _GC, _GC)], idxs[b])
            copies[b] = pltpu.async_copy(x_hbm.at[idxs[b]], bufs[b], sems[b])
            if i > 0:
                pb = (i - 1) % 2
                copies[pb].wait()
                pltpu.sync_copy(bufs[pb],
                                xs_hbm.at[pl.ds(base + (i - 1) * _GC, _GC)])
        lb = (_GN - 1) % 2
        copies[lb].wait()
        pltpu.sync_copy(bufs[lb],
                        xs_hbm.at[pl.ds(base + (_GN - 1) * _GC, _GC)])

    return gather_k


def _make_combine():
    TPW = T // _NW

    @functools.partial(
        pl.kernel,
        out_type=jax.ShapeDtypeStruct((T, H), jnp.float32),
        mesh=_sc_mesh(),
        scratch_types=[
            pltpu.VMEM((TPW * K,), jnp.int32),
            pltpu.VMEM((K, H), jnp.float32),
            pltpu.VMEM((K, H), jnp.float32),
            pltpu.VMEM((H,), jnp.float32),
            pltpu.SemaphoreType.DMA,
            pltpu.SemaphoreType.DMA,
        ],
    )
    def combine_k(part_hbm, pos_hbm, out_hbm, idx_all, rows0, rows1, orow_v,
                  sem0, sem1):
        wid = lax.axis_index("s") * 2 + lax.axis_index("c")
        tbase = wid * TPW
        pltpu.sync_copy(pos_hbm.at[pl.ds(tbase * K, TPW * K)], idx_all)
        rows = (rows0, rows1)
        sems = (sem0, sem1)
        cps = [None, None]
        cps[0] = pltpu.async_copy(
            part_hbm.at[idx_all.at[pl.ds(0, K)]], rows0, sem0)
        for t in range(TPW):
            b = t % 2
            if t + 1 < TPW:
                nb = (t + 1) % 2
                cps[nb] = pltpu.async_copy(
                    part_hbm.at[idx_all.at[pl.ds((t + 1) * K, K)]],
                    rows[nb], sems[nb])
            cps[b].wait()

            def chunk(c, carry2, _b=b):
                s = c * 16
                acc = rows[_b][0, pl.ds(s, 16)]
                for j in range(1, K):
                    acc = acc + rows[_b][j, pl.ds(s, 16)]
                orow_v[pl.ds(s, 16)] = acc
                return carry2

            lax.fori_loop(0, H // 16, chunk, 0)
            pltpu.sync_copy(orow_v, out_hbm.at[tbase + t])

    return combine_k


def _gemm_body(be_ref, bv_ref, xs_ref, ws_ref, gup_ref, dwn_ref, out_ref):
    b = pl.program_id(0)

    @pl.when(bv_ref[b] == 1)
    def _():
        xb = xs_ref[...]
        w1 = gup_ref[0].astype(jnp.bfloat16)
        gu = lax.dot_general(xb, w1, (((1,), (1,)), ((), ())),
                             preferred_element_type=jnp.float32)
        g = gu[:, :I]
        u = gu[:, I:]
        act = (g / (1.0 + jnp.exp(-g))) * u
        act = act * ws_ref[0, 0, :][:, None]
        w2 = dwn_ref[0].astype(jnp.bfloat16)
        out_ref[...] = lax.dot_general(act.astype(jnp.bfloat16), w2,
                                       (((1,), (1,)), ((), ())),
                                       preferred_element_type=jnp.float32)


def _gemm(xs, ws3, gate_up_w, down_w, be, bv, interpret=False):
    grid_spec = pltpu.PrefetchScalarGridSpec(
        num_scalar_prefetch=2,
        grid=(NB,),
        in_specs=[
            pl.BlockSpec((M, H), lambda b, be, bv: (b, 0)),
            pl.BlockSpec((1, 1, M), lambda b, be, bv: (b, 0, 0)),
            pl.BlockSpec((1, 2 * I, H), lambda b, be, bv: (be[b], 0, 0)),
            pl.BlockSpec((1, H, I), lambda b, be, bv: (be[b], 0, 0)),
        ],
        out_specs=pl.BlockSpec((M, H), lambda b, be, bv: (b, 0)),
    )
    return pl.pallas_call(
        _gemm_body,
        grid_spec=grid_spec,
        out_shape=jax.ShapeDtypeStruct((PP, H), jnp.float32),
        compiler_params=pltpu.CompilerParams(
            dimension_semantics=("arbitrary",)),
        interpret=interpret,
    )(be, bv, xs, ws3, gate_up_w, down_w)


def kernel(hidden_states, gate_w, gate_up_w, down_w):
    x = hidden_states.reshape(T, H)
    w, sel = _route(x, gate_w)
    pos, be, bv = _plan(sel)
    be = be.reshape(NB)
    bv = bv.reshape(NB)
    pflat = pos.reshape(P)
    tok_slot = jnp.zeros((PP,), jnp.int32).at[pflat].set(
        jnp.arange(P, dtype=jnp.int32) // K)
    ws = jnp.zeros((PP,), jnp.float32).at[pflat].set(w.reshape(P))
    x16 = x.astype(jnp.bfloat16)
    xs = jnp.take(x16, tok_slot, axis=0)
    partial = _gemm(xs, ws.reshape(NB, 1, M), gate_up_w, down_w, be, bv)
    out = _make_combine()(partial, pflat)
    return out.reshape(1, T, H)


# speedup vs baseline: 1.0861x; 1.0861x over previous
import functools

import jax
import jax.numpy as jnp
from jax import lax
from jax.experimental import pallas as pl
from jax.experimental.pallas import tpu as pltpu
from jax.experimental.pallas import tpu_sc as plsc

H = 2048
I = 768
E = 64
K = 8
T = 2048
P = T * K
M = 256
NB = P // M + E
PP = NB * M


def _router_body(x_ref, gw_ref, w_ref, s_ref):
    x = x_ref[...]
    gw = gw_ref[...]
    logits = lax.dot_general(x, gw, (((1,), (1,)), ((), ())),
                             preferred_element_type=jnp.float32)
    m = jnp.max(logits, axis=1, keepdims=True)
    p = jnp.exp(logits - m)
    p = p / jnp.sum(p, axis=1, keepdims=True)
    iota = lax.broadcasted_iota(jnp.int32, p.shape, 1)
    vals = []
    idxs = []
    for _ in range(K):
        mk = jnp.max(p, axis=1, keepdims=True)
        amk = jnp.min(jnp.where(p == mk, iota, E), axis=1, keepdims=True)
        vals.append(mk)
        idxs.append(amk)
        p = jnp.where(iota == amk, -1.0, p)
    w8 = jnp.concatenate(vals, axis=1)
    s8 = jnp.concatenate(idxs, axis=1)
    w_ref[...] = w8 / jnp.sum(w8, axis=1, keepdims=True)
    s_ref[...] = s8


def _route(x, gate_w, interpret=False):
    RB = 256
    return pl.pallas_call(
        _router_body,
        grid=(T // RB,),
        in_specs=[
            pl.BlockSpec((RB, H), lambda b: (b, 0)),
            pl.BlockSpec((E, H), lambda b: (0, 0)),
        ],
        out_specs=[
            pl.BlockSpec((RB, K), lambda b: (b, 0)),
            pl.BlockSpec((RB, K), lambda b: (b, 0)),
        ],
        out_shape=[
            jax.ShapeDtypeStruct((T, K), jnp.float32),
            jax.ShapeDtypeStruct((T, K), jnp.int32),
        ],
        interpret=interpret,
    )(x, gate_w)


def _plan_body(sel_ref, pos_ref, be_ref, bv_ref):
    iot = lax.broadcasted_iota(jnp.int32, (T, E), 1)
    carry = jnp.zeros((1, E), jnp.float32)
    Os = []
    incls = []
    for k in range(K):
        col = sel_ref[:, k:k + 1]
        O = (col == iot).astype(jnp.float32)
        incl = O
        sh = 1
        while sh < T:
            incl = incl + jnp.concatenate(
                [jnp.zeros((sh, E), jnp.float32), incl[:T - sh]], axis=0)
            sh *= 2
        incl = incl + carry
        Os.append(O)
        incls.append(incl)
        carry = incl[T - 1:T, :]
    counts = carry
    nblk = jnp.floor((counts + (M - 1)) / M)
    trili = (lax.broadcasted_iota(jnp.int32, (E, E), 0)
             <= lax.broadcasted_iota(jnp.int32, (E, E), 1)).astype(jnp.float32)
    cumblk = lax.dot_general(nblk, trili, (((1,), (0,)), ((), ())),
                             preferred_element_type=jnp.float32)
    pad_start = (cumblk - nblk) * M
    for k in range(K):
        slot = jnp.sum((incls[k] - 1.0 + pad_start) * Os[k],
                       axis=1, keepdims=True)
        pos_ref[:, k:k + 1] = slot.astype(jnp.int32)
    total = jnp.sum(nblk)
    biot = lax.broadcasted_iota(jnp.int32, (NB, E), 0).astype(jnp.float32)
    cb = jnp.broadcast_to(cumblk, (NB, E))
    be_raw = jnp.sum((cb <= biot).astype(jnp.int32), axis=1, keepdims=True)
    be_c = jnp.minimum(be_raw, E - 1)
    bvec = lax.broadcasted_iota(jnp.int32, (NB, 1), 0).astype(jnp.float32)
    bv = (bvec < total).astype(jnp.int32)
    last_e = jnp.sum(jnp.where(bvec == total - 1.0, be_c, 0))
    be_ref[...] = jnp.where(bv == 1, be_c, last_e)
    bv_ref[...] = bv


def _plan(sel, interpret=False):
    return pl.pallas_call(
        _plan_body,
        out_shape=[
            jax.ShapeDtypeStruct((T, K), jnp.int32),
            jax.ShapeDtypeStruct((NB, 1), jnp.int32),
            jax.ShapeDtypeStruct((NB, 1), jnp.int32),
        ],
        interpret=interpret,
    )(sel)


_NW = 32
_GC = 32
_GN = PP // _NW // _GC


def _sc_mesh():
    return plsc.VectorSubcoreMesh(core_axis_name="c", subcore_axis_name="s")


def _make_gather():
    HW = H // 2

    @functools.partial(
        pl.kernel,
        out_type=jax.ShapeDtypeStruct((PP, HW), jnp.int32),
        mesh=_sc_mesh(),
        scratch_types=[
            pltpu.VMEM((_GC,), jnp.int32),
            pltpu.VMEM((_GC,), jnp.int32),
            pltpu.VMEM((_GC, HW), jnp.int32),
            pltpu.VMEM((_GC, HW), jnp.int32),
            pltpu.SemaphoreType.DMA,
            pltpu.SemaphoreType.DMA,
        ],
    )
    def gather_k(x_hbm, tok_hbm, xs_hbm, idx0, idx1, buf0, buf1, sem0, sem1):
        wid = lax.axis_index("s") * 2 + lax.axis_index("c")
        base = wid * (PP // _NW)
        idxs = (idx0, idx1)
        bufs = (buf0, buf1)
        sems = (sem0, sem1)
        copies = [None, None]
        for i in range(_GN):
            b = i % 2
            pltpu.sync_copy(tok_hbm.at[pl.ds(base + i * _GC, _GC)], idxs[b])
            copies[b] = pltpu.async_copy(x_hbm.at[idxs[b]], bufs[b], sems[b])
            if i > 0:
                pb = (i - 1) % 2
                copies[pb].wait()
                pltpu.sync_copy(bufs[pb],
                                xs_hbm.at[pl.ds(base + (i - 1) * _GC, _GC)])
        lb = (_GN - 1) % 2
        copies[lb].wait()
        pltpu.sync_copy(bufs[lb],
                        xs_hbm.at[pl.ds(base + (_GN - 1) * _GC, _GC)])

    return gather_k


def _make_combine():
    TPW = T // _NW

    @functools.partial(
        pl.kernel,
        out_type=jax.ShapeDtypeStruct((T, H), jnp.float32),
        mesh=_sc_mesh(),
        scratch_types=[
            pltpu.VMEM((TPW * K,), jnp.int32),
            pltpu.VMEM((K, H), jnp.float32),
            pltpu.VMEM((K, H), jnp.float32),
            pltpu.VMEM((H,), jnp.float32),
            pltpu.SemaphoreType.DMA,
            pltpu.SemaphoreType.DMA,
        ],
    )
    def combine_k(part_hbm, pos_hbm, out_hbm, idx_all, rows0, rows1, orow_v,
                  sem0, sem1):
        wid = lax.axis_index("s") * 2 + lax.axis_index("c")
        tbase = wid * TPW
        pltpu.sync_copy(pos_hbm.at[pl.ds(tbase * K, TPW * K)], idx_all)
        rows = (rows0, rows1)
        sems = (sem0, sem1)
        cps = [None, None]
        cps[0] = pltpu.async_copy(
            part_hbm.at[idx_all.at[pl.ds(0, K)]], rows0, sem0)
        for t in range(TPW):
            b = t % 2
            if t + 1 < TPW:
                nb = (t + 1) % 2
                cps[nb] = pltpu.async_copy(
                    part_hbm.at[idx_all.at[pl.ds((t + 1) * K, K)]],
                    rows[nb], sems[nb])
            cps[b].wait()

            def chunk(c, carry2, _b=b):
                s = c * 16
                acc = rows[_b][0, pl.ds(s, 16)]
                for j in range(1, K):
                    acc = acc + rows[_b][j, pl.ds(s, 16)]
                orow_v[pl.ds(s, 16)] = acc
                return carry2

            lax.fori_loop(0, H // 16, chunk, 0)
            pltpu.sync_copy(orow_v, out_hbm.at[tbase + t])

    return combine_k


def _gemm_body(be_ref, bv_ref, xs_ref, ws_ref, gup_ref, dwn_ref, out_ref,
               w1c_ref, w2c_ref):
    b = pl.program_id(0)
    valid = bv_ref[b] == 1
    new_w = jnp.logical_or(b == 0,
                           be_ref[b] != be_ref[jnp.maximum(b - 1, 0)])

    @pl.when(jnp.logical_and(valid, new_w))
    def _():
        w1c_ref[...] = gup_ref[0].astype(jnp.bfloat16)
        w2c_ref[...] = dwn_ref[0].astype(jnp.bfloat16)

    @pl.when(valid)
    def _():
        xb = xs_ref[...]
        gu = lax.dot_general(xb, w1c_ref[...], (((1,), (1,)), ((), ())),
                             preferred_element_type=jnp.float32)
        g = gu[:, :I]
        u = gu[:, I:]
        act = (g / (1.0 + jnp.exp(-g))) * u
        act = act * ws_ref[0, 0, :][:, None]
        out_ref[...] = lax.dot_general(act.astype(jnp.bfloat16), w2c_ref[...],
                                       (((1,), (1,)), ((), ())),
                                       preferred_element_type=jnp.float32)


def _gemm(xs, ws3, gate_up_w, down_w, be, bv, interpret=False):
    grid_spec = pltpu.PrefetchScalarGridSpec(
        num_scalar_prefetch=2,
        grid=(NB,),
        in_specs=[
            pl.BlockSpec((M, H), lambda b, be, bv: (b, 0)),
            pl.BlockSpec((1, 1, M), lambda b, be, bv: (b, 0, 0)),
            pl.BlockSpec((1, 2 * I, H), lambda b, be, bv: (be[b], 0, 0)),
            pl.BlockSpec((1, H, I), lambda b, be, bv: (be[b], 0, 0)),
        ],
        out_specs=pl.BlockSpec((M, H), lambda b, be, bv: (b, 0)),
        scratch_shapes=[
            pltpu.VMEM((2 * I, H), jnp.bfloat16),
            pltpu.VMEM((H, I), jnp.bfloat16),
        ],
    )
    return pl.pallas_call(
        _gemm_body,
        grid_spec=grid_spec,
        out_shape=jax.ShapeDtypeStruct((PP, H), jnp.float32),
        compiler_params=pltpu.CompilerParams(
            dimension_semantics=("arbitrary",)),
        interpret=interpret,
    )(be, bv, xs, ws3, gate_up_w, down_w)


def kernel(hidden_states, gate_w, gate_up_w, down_w):
    x = hidden_states.reshape(T, H)
    w, sel = _route(x, gate_w)
    pos, be, bv = _plan(sel)
    be = be.reshape(NB)
    bv = bv.reshape(NB)
    pflat = pos.reshape(P)
    tok_slot = jnp.zeros((PP,), jnp.int32).at[pflat].set(
        jnp.arange(P, dtype=jnp.int32) // K)
    ws = jnp.zeros((PP,), jnp.float32).at[pflat].set(w.reshape(P))
    x16 = x.astype(jnp.bfloat16)
    xs = jnp.take(x16, tok_slot, axis=0)
    partial = _gemm(xs, ws.reshape(NB, 1, M), gate_up_w, down_w, be, bv)
    out = _make_combine()(partial, pflat)
    return out.reshape(1, T, H)


# speedup vs baseline: 1.1193x; 1.0306x over previous
import functools

import jax
import jax.numpy as jnp
from jax import lax
from jax.experimental import pallas as pl
from jax.experimental.pallas import tpu as pltpu
from jax.experimental.pallas import tpu_sc as plsc

H = 2048
I = 768
E = 64
K = 8
T = 2048
P = T * K
M = 256
NB = P // M + E
PP = NB * M


def _router_body(x_ref, gw_ref, w_ref, s_ref, x16_ref):
    x = x_ref[...]
    x16_ref[...] = x.astype(jnp.bfloat16)
    gw = gw_ref[...]
    logits = lax.dot_general(x, gw, (((1,), (1,)), ((), ())),
                             preferred_element_type=jnp.float32)
    m = jnp.max(logits, axis=1, keepdims=True)
    p = jnp.exp(logits - m)
    p = p / jnp.sum(p, axis=1, keepdims=True)
    iota = lax.broadcasted_iota(jnp.int32, p.shape, 1)
    vals = []
    idxs = []
    for _ in range(K):
        mk = jnp.max(p, axis=1, keepdims=True)
        amk = jnp.min(jnp.where(p == mk, iota, E), axis=1, keepdims=True)
        vals.append(mk)
        idxs.append(amk)
        p = jnp.where(iota == amk, -1.0, p)
    w8 = jnp.concatenate(vals, axis=1)
    s8 = jnp.concatenate(idxs, axis=1)
    w_ref[...] = w8 / jnp.sum(w8, axis=1, keepdims=True)
    s_ref[...] = s8


def _route(x, gate_w, interpret=False):
    RB = 256
    return pl.pallas_call(
        _router_body,
        grid=(T // RB,),
        in_specs=[
            pl.BlockSpec((RB, H), lambda b: (b, 0)),
            pl.BlockSpec((E, H), lambda b: (0, 0)),
        ],
        out_specs=[
            pl.BlockSpec((RB, K), lambda b: (b, 0)),
            pl.BlockSpec((RB, K), lambda b: (b, 0)),
            pl.BlockSpec((RB, H), lambda b: (b, 0)),
        ],
        out_shape=[
            jax.ShapeDtypeStruct((T, K), jnp.float32),
            jax.ShapeDtypeStruct((T, K), jnp.int32),
            jax.ShapeDtypeStruct((T, H), jnp.bfloat16),
        ],
        interpret=interpret,
    )(x, gate_w)


def _plan_body(sel_ref, pos_ref, be_ref, bv_ref):
    iot = lax.broadcasted_iota(jnp.int32, (T, E), 1)
    carry = jnp.zeros((1, E), jnp.float32)
    Os = []
    incls = []
    for k in range(K):
        col = sel_ref[:, k:k + 1]
        O = (col == iot).astype(jnp.float32)
        incl = O
        sh = 1
        while sh < T:
            incl = incl + jnp.concatenate(
                [jnp.zeros((sh, E), jnp.float32), incl[:T - sh]], axis=0)
            sh *= 2
        incl = incl + carry
        Os.append(O)
        incls.append(incl)
        carry = incl[T - 1:T, :]
    counts = carry
    nblk = jnp.floor((counts + (M - 1)) / M)
    trili = (lax.broadcasted_iota(jnp.int32, (E, E), 0)
             <= lax.broadcasted_iota(jnp.int32, (E, E), 1)).astype(jnp.float32)
    cumblk = lax.dot_general(nblk, trili, (((1,), (0,)), ((), ())),
                             preferred_element_type=jnp.float32)
    pad_start = (cumblk - nblk) * M
    for k in range(K):
        slot = jnp.sum((incls[k] - 1.0 + pad_start) * Os[k],
                       axis=1, keepdims=True)
        pos_ref[:, k:k + 1] = slot.astype(jnp.int32)
    total = jnp.sum(nblk)
    biot = lax.broadcasted_iota(jnp.int32, (NB, E), 0).astype(jnp.float32)
    cb = jnp.broadcast_to(cumblk, (NB, E))
    be_raw = jnp.sum((cb <= biot).astype(jnp.int32), axis=1, keepdims=True)
    be_c = jnp.minimum(be_raw, E - 1)
    bvec = lax.broadcasted_iota(jnp.int32, (NB, 1), 0).astype(jnp.float32)
    bv = (bvec < total).astype(jnp.int32)
    last_e = jnp.sum(jnp.where(bvec == total - 1.0, be_c, 0))
    be_ref[...] = jnp.where(bv == 1, be_c, last_e)
    bv_ref[...] = bv


def _plan(sel, interpret=False):
    return pl.pallas_call(
        _plan_body,
        out_shape=[
            jax.ShapeDtypeStruct((T, K), jnp.int32),
            jax.ShapeDtypeStruct((NB, 1), jnp.int32),
            jax.ShapeDtypeStruct((NB, 1), jnp.int32),
        ],
        interpret=interpret,
    )(sel)


_NW = 32


def _sc_mesh():
    return plsc.VectorSubcoreMesh(core_axis_name="c", subcore_axis_name="s")


def _make_combine():
    TPW = T // _NW

    @functools.partial(
        pl.kernel,
        out_type=jax.ShapeDtypeStruct((T, H), jnp.float32),
        mesh=_sc_mesh(),
        scratch_types=[
            pltpu.VMEM((TPW * K,), jnp.int32),
            pltpu.VMEM((K, H), jnp.float32),
            pltpu.VMEM((K, H), jnp.float32),
            pltpu.VMEM((H,), jnp.float32),
            pltpu.SemaphoreType.DMA,
            pltpu.SemaphoreType.DMA,
        ],
    )
    def combine_k(part_hbm, pos_hbm, out_hbm, idx_all, rows0, rows1, orow_v,
                  sem0, sem1):
        wid = lax.axis_index("s") * 2 + lax.axis_index("c")
        tbase = wid * TPW
        pltpu.sync_copy(pos_hbm.at[pl.ds(tbase * K, TPW * K)], idx_all)
        rows = (rows0, rows1)
        sems = (sem0, sem1)
        cps = [None, None]
        cps[0] = pltpu.async_copy(
            part_hbm.at[idx_all.at[pl.ds(0, K)]], rows0, sem0)
        for t in range(TPW):
            b = t % 2
            if t + 1 < TPW:
                nb = (t + 1) % 2
                cps[nb] = pltpu.async_copy(
                    part_hbm.at[idx_all.at[pl.ds((t + 1) * K, K)]],
                    rows[nb], sems[nb])
            cps[b].wait()

            def chunk(c, carry2, _b=b):
                s = c * 16
                acc = rows[_b][0, pl.ds(s, 16)]
                for j in range(1, K):
                    acc = acc + rows[_b][j, pl.ds(s, 16)]
                orow_v[pl.ds(s, 16)] = acc
                return carry2

            lax.fori_loop(0, H // 16, chunk, 0)
            pltpu.sync_copy(orow_v, out_hbm.at[tbase + t])

    return combine_k


def _gemm_body(be_ref, bv_ref, xs_ref, ws_ref, gup_ref, dwn_ref, out_ref):
    b = pl.program_id(0)

    @pl.when(bv_ref[b] == 1)
    def _():
        xb = xs_ref[...]
        w1 = gup_ref[0].astype(jnp.bfloat16)
        gu = lax.dot_general(xb, w1, (((1,), (1,)), ((), ())),
                             preferred_element_type=jnp.float32)
        g = gu[:, :I]
        u = gu[:, I:]
        act = (g / (1.0 + jnp.exp(-g))) * u
        act = act * ws_ref[0, 0, :][:, None]
        w2 = dwn_ref[0].astype(jnp.bfloat16)
        out_ref[...] = lax.dot_general(act.astype(jnp.bfloat16), w2,
                                       (((1,), (1,)), ((), ())),
                                       preferred_element_type=jnp.float32)


def _gemm(xs, ws3, gate_up_w, down_w, be, bv, interpret=False):
    grid_spec = pltpu.PrefetchScalarGridSpec(
        num_scalar_prefetch=2,
        grid=(NB,),
        in_specs=[
            pl.BlockSpec((M, H), lambda b, be, bv: (b, 0)),
            pl.BlockSpec((1, 1, M), lambda b, be, bv: (b, 0, 0)),
            pl.BlockSpec((1, 2 * I, H), lambda b, be, bv: (be[b], 0, 0)),
            pl.BlockSpec((1, H, I), lambda b, be, bv: (be[b], 0, 0)),
        ],
        out_specs=pl.BlockSpec((M, H), lambda b, be, bv: (b, 0)),
    )
    return pl.pallas_call(
        _gemm_body,
        grid_spec=grid_spec,
        out_shape=jax.ShapeDtypeStruct((PP, H), jnp.float32),
        compiler_params=pltpu.CompilerParams(
            dimension_semantics=("arbitrary",)),
        interpret=interpret,
    )(be, bv, xs, ws3, gate_up_w, down_w)


def kernel(hidden_states, gate_w, gate_up_w, down_w):
    x = hidden_states.reshape(T, H)
    w, sel, x16 = _route(x, gate_w)
    pos, be, bv = _plan(sel)
    be = be.reshape(NB)
    bv = bv.reshape(NB)
    pflat = pos.reshape(P)
    tok_slot = jnp.zeros((PP,), jnp.int32).at[pflat].set(
        jnp.arange(P, dtype=jnp.int32) // K)
    ws = jnp.zeros((PP,), jnp.float32).at[pflat].set(w.reshape(P))
    xs = jnp.take(x16, tok_slot, axis=0)
    partial = _gemm(xs, ws.reshape(NB, 1, M), gate_up_w, down_w, be, bv)
    out = _make_combine()(partial, pflat)
    return out.reshape(1, T, H)
